# Initial kernel scaffold; baseline (speedup 1.0000x reference)
import jax
import jax.numpy as jnp
from jax.experimental import pallas as pl


def kernel(x, edge_index, params):
    raise NotImplementedError("write your pallas kernel here")


# speedup vs baseline: 2.0491x; 2.0491x over previous
import functools

import jax
import jax.numpy as jnp
from jax import lax
from jax.experimental import pallas as pl
from jax.experimental.pallas import tpu as pltpu
from jax.experimental.pallas import tpu_sc as plsc

N = 10000
E = 320000
EP = E + 256
RNG = 48
NRANGES = 224
RPW = 7
NOUT = NRANGES * RNG
STN = 240

_G_CFG = {128: 128, 512: 64, 256: 128}


def _mm_bias_relu(x, W, b, di):
    BN = 1000

    def body(xr, wr, br, outr):
        outr[...] = jnp.maximum(
            jnp.dot(xr[...], wr[...], preferred_element_type=jnp.float32)
            + br[...], 0.0)

    return pl.pallas_call(
        body,
        grid=(N // BN,),
        in_specs=[
            pl.BlockSpec((BN, di), lambda i: (i, 0)),
            pl.BlockSpec((di, di), lambda i: (0, 0)),
            pl.BlockSpec((1, di), lambda i: (0, 0)),
        ],
        out_specs=pl.BlockSpec((BN, di), lambda i: (i, 0)),
        out_shape=jax.ShapeDtypeStruct((N, di), jnp.float32),
    )(x, W, b)


def _combine(numden, x, Wl, Wrl, bias, di, ho, relu):
    BN = 1000

    def body(ndr, xr, wlr, wrlr, br, outr):
        nd = ndr[...]
        num = nd[:, :di]
        den = nd[:, di:]
        aggr = num / (den + 1e-16)
        h = (jnp.dot(aggr, wlr[...], preferred_element_type=jnp.float32)
             + jnp.dot(xr[...], wrlr[...], preferred_element_type=jnp.float32)
             + br[...])
        outr[...] = jnp.maximum(h, 0.0) if relu else h

    return pl.pallas_call(
        body,
        grid=(N // BN,),
        in_specs=[
            pl.BlockSpec((BN, 2 * di), lambda i: (i, 0)),
            pl.BlockSpec((BN, di), lambda i: (i, 0)),
            pl.BlockSpec((di, ho), lambda i: (0, 0)),
            pl.BlockSpec((di, ho), lambda i: (0, 0)),
            pl.BlockSpec((1, ho), lambda i: (0, 0)),
        ],
        out_specs=pl.BlockSpec((BN, ho), lambda i: (i, 0)),
        out_shape=jax.ShapeDtypeStruct((N, ho), jnp.float32),
    )(numden, x, Wl, Wrl, bias)


def _edge_softmax_aggr(srcp, dstp, xp, t, starts, di):
    G = _G_CFG[di]
    NV = di // 16
    mesh = plsc.VectorSubcoreMesh(core_axis_name="c", subcore_axis_name="s")

    @functools.partial(
        pl.kernel,
        mesh=mesh,
        compiler_params=pltpu.CompilerParams(needs_layout_passes=False),
        out_type=jax.ShapeDtypeStruct((NOUT, 2 * di), jnp.float32),
        scratch_types=[
            pltpu.SMEM((STN,), jnp.int32),
            pltpu.VMEM((STN,), jnp.int32),
            pltpu.VMEM((G,), jnp.int32),
            pltpu.SMEM((G,), jnp.int32),
            pltpu.VMEM((G,), jnp.int32),
            pltpu.VMEM((G, di), jnp.float32),
            pltpu.VMEM((di,), jnp.float32),
            pltpu.VMEM((RNG + 8, 2 * di), jnp.float32),
            pltpu.SemaphoreType.DMA,
        ],
    )
    def k(srcp_hbm, dstp_hbm, xp_hbm, t_hbm, st_hbm, out_hbm,
          stv, stv_v, gsrc, gdstb, gdst_v, rows_in, tvec, acc, sem):
        c = lax.axis_index("c")
        s = lax.axis_index("s")
        w = s * 2 + c
        zf = jnp.zeros((16,), jnp.float32)
        pltpu.sync_copy(t_hbm, tvec)
        pltpu.sync_copy(st_hbm, stv_v)
        for kq in range(STN // 16):
            vv = stv_v[pl.ds(kq * 16, 16)]
            for i in range(16):
                stv[kq * 16 + i] = vv[i]

        for j in range(RPW):
            r = w * RPW + j
            s0 = stv[r]
            s1 = stv[r + 1]
            st8 = (s0 // 8) * 8
            nch = (s1 - st8 + G - 1) // G
            node0 = r * RNG

            def zacc_body(rr2, _):
                for kk in range(2 * NV):
                    acc[rr2, pl.ds(kk * 16, 16)] = zf
                return 0
            lax.fori_loop(0, RNG, zacc_body, 0)

            def chunk_body(g, _):
                off = st8 + g * G
                pltpu.sync_copy(srcp_hbm.at[pl.ds(off, G)], gsrc)
                pltpu.sync_copy(dstp_hbm.at[pl.ds(off, G)], gdst_v)
                for kq in range(G // 16):
                    vv = gdst_v[pl.ds(kq * 16, 16)]
                    for i in range(16):
                        gdstb[kq * 16 + i] = vv[i]
                pltpu.async_copy(xp_hbm.at[gsrc], rows_in, sem).wait()

                def row_body(rr, _2):
                    dl = gdstb[rr] - node0
                    idxr = jnp.where((dl >= 0) & (dl < RNG), dl, RNG)
                    for kk in range(NV):
                        v = rows_in[rr, pl.ds(kk * 16, 16)]
                        tv = tvec[pl.ds(kk * 16, 16)]
                        e = jnp.exp(v * tv)
                        an = acc[idxr, pl.ds(kk * 16, 16)]
                        acc[idxr, pl.ds(kk * 16, 16)] = an + e * v
                        ad = acc[idxr, pl.ds(di + kk * 16, 16)]
                        acc[idxr, pl.ds(di + kk * 16, 16)] = ad + e
                    return 0
                lax.fori_loop(0, G, row_body, 0)
                return 0
            lax.fori_loop(0, nch, chunk_body, 0)

            for z in range(RNG // 8):
                pltpu.sync_copy(acc.at[pl.ds(z * 8, 8)],
                                out_hbm.at[pl.ds(node0 + z * 8, 8)])

    return k(srcp, dstp, xp, t, starts)


def kernel(x, edge_index, params):
    src = edge_index[0]
    dst = edge_index[1]
    order = jnp.argsort(dst)
    dsts = dst[order]
    srcp = jnp.concatenate([src[order], jnp.zeros((EP - E,), jnp.int32)])
    dstp = jnp.concatenate([dsts, jnp.full((EP - E,), N, jnp.int32)])
    bounds = jnp.arange(0, (NRANGES + 1) * RNG, RNG)
    starts = jnp.searchsorted(dsts, bounds).astype(jnp.int32)
    starts = jnp.concatenate(
        [starts, jnp.full((STN - NRANGES - 1,), E, jnp.int32)])

    h = x
    dims = [(128, 512), (512, 256), (256, 128)]
    for l, (di, ho) in enumerate(dims):
        xp = _mm_bias_relu(h, params[f"Wp{l}"],
                           params[f"bp{l}"].reshape(1, di), di)
        numden = _edge_softmax_aggr(srcp, dstp, xp,
                                    params[f"t{l}"].reshape(di), starts, di)
        Wrl = params[f"Wr{l}"] + params[f"Wlin{l}"]
        bias = (params[f"bl{l}"] + params[f"blin{l}"]).reshape(1, ho)
        h = _combine(numden, h, params[f"Wl{l}"], Wrl, bias, di, ho,
                     relu=(l < 2))
    return h


# speedup vs baseline: 2.0958x; 1.0228x over previous
import functools

import jax
import jax.numpy as jnp
from jax import lax
from jax.experimental import pallas as pl
from jax.experimental.pallas import tpu as pltpu
from jax.experimental.pallas import tpu_sc as plsc

N = 10000
E = 320000
EP = E + 256
RNG = 48
NRANGES = 224
RPW = 7
NOUT = NRANGES * RNG
STN = 240

_G_CFG = {128: 128, 512: 96, 256: 128}


def _mm_bias_relu(x, W, b, di):
    BN = 1000

    def body(xr, wr, br, outr):
        outr[...] = jnp.maximum(
            jnp.dot(xr[...], wr[...], preferred_element_type=jnp.float32)
            + br[...], 0.0)

    return pl.pallas_call(
        body,
        grid=(N // BN,),
        in_specs=[
            pl.BlockSpec((BN, di), lambda i: (i, 0)),
            pl.BlockSpec((di, di), lambda i: (0, 0)),
            pl.BlockSpec((1, di), lambda i: (0, 0)),
        ],
        out_specs=pl.BlockSpec((BN, di), lambda i: (i, 0)),
        out_shape=jax.ShapeDtypeStruct((N, di), jnp.float32),
    )(x, W, b)


def _combine(numden, x, Wl, Wrl, bias, di, ho, relu):
    BN = 1000

    def body(ndr, xr, wlr, wrlr, br, outr):
        nd = ndr[...]
        num = nd[:, :di]
        den = nd[:, di:]
        aggr = num / (den + 1e-16)
        h = (jnp.dot(aggr, wlr[...], preferred_element_type=jnp.float32)
             + jnp.dot(xr[...], wrlr[...], preferred_element_type=jnp.float32)
             + br[...])
        outr[...] = jnp.maximum(h, 0.0) if relu else h

    return pl.pallas_call(
        body,
        grid=(N // BN,),
        in_specs=[
            pl.BlockSpec((BN, 2 * di), lambda i: (i, 0)),
            pl.BlockSpec((BN, di), lambda i: (i, 0)),
            pl.BlockSpec((di, ho), lambda i: (0, 0)),
            pl.BlockSpec((di, ho), lambda i: (0, 0)),
            pl.BlockSpec((1, ho), lambda i: (0, 0)),
        ],
        out_specs=pl.BlockSpec((BN, ho), lambda i: (i, 0)),
        out_shape=jax.ShapeDtypeStruct((N, ho), jnp.float32),
    )(numden, x, Wl, Wrl, bias)


def _edge_softmax_aggr(srcp, dstp, xp, t, starts, di):
    G = _G_CFG[di]
    NV = di // 16
    mesh = plsc.VectorSubcoreMesh(core_axis_name="c", subcore_axis_name="s")

    @functools.partial(
        pl.kernel,
        mesh=mesh,
        compiler_params=pltpu.CompilerParams(needs_layout_passes=False),
        out_type=jax.ShapeDtypeStruct((NOUT, 2 * di), jnp.float32),
        scratch_types=[
            pltpu.SMEM((STN,), jnp.int32),
            pltpu.VMEM((STN,), jnp.int32),
            pltpu.VMEM((G,), jnp.int32),
            pltpu.SMEM((G,), jnp.int32),
            pltpu.VMEM((G,), jnp.int32),
            pltpu.VMEM((G, di), jnp.float32),
            pltpu.VMEM((di,), jnp.float32),
            pltpu.VMEM((RNG + 8, 2 * di), jnp.float32),
            pltpu.SemaphoreType.DMA,
        ],
    )
    def k(srcp_hbm, dstp_hbm, xp_hbm, t_hbm, st_hbm, out_hbm,
          stv, stv_v, gsrc, gdstb, gdst_v, rows_in, tvec, acc, sem):
        c = lax.axis_index("c")
        s = lax.axis_index("s")
        w = s * 2 + c
        zf = jnp.zeros((16,), jnp.float32)
        pltpu.sync_copy(t_hbm, tvec)
        pltpu.sync_copy(st_hbm, stv_v)
        for kq in range(STN // 16):
            vv = stv_v[pl.ds(kq * 16, 16)]
            for i in range(16):
                stv[kq * 16 + i] = vv[i]

        for j in range(RPW):
            r = w * RPW + j
            s0 = stv[r]
            s1 = stv[r + 1]
            st8 = (s0 // 8) * 8
            nch = (s1 - st8 + G - 1) // G
            node0 = r * RNG

            def zacc_body(rr2, _):
                for kk in range(2 * NV):
                    acc[rr2, pl.ds(kk * 16, 16)] = zf
                return 0
            lax.fori_loop(0, RNG, zacc_body, 0)

            def chunk_body(g, _):
                off = st8 + g * G
                pltpu.sync_copy(srcp_hbm.at[pl.ds(off, G)], gsrc)
                pltpu.sync_copy(dstp_hbm.at[pl.ds(off, G)], gdst_v)
                for kq in range(G // 16):
                    vv = gdst_v[pl.ds(kq * 16, 16)]
                    dlv = vv - node0
                    idxv = jnp.where((dlv >= 0) & (dlv < RNG), dlv, RNG)
                    for i in range(16):
                        gdstb[kq * 16 + i] = idxv[i]
                pltpu.async_copy(xp_hbm.at[gsrc], rows_in, sem).wait()

                def row_body(rr, _2):
                    idxr = gdstb[rr]
                    for kk in range(NV):
                        v = rows_in[rr, pl.ds(kk * 16, 16)]
                        tv = tvec[pl.ds(kk * 16, 16)]
                        e = jnp.exp(v * tv)
                        an = acc[idxr, pl.ds(kk * 16, 16)]
                        acc[idxr, pl.ds(kk * 16, 16)] = an + e * v
                        ad = acc[idxr, pl.ds(di + kk * 16, 16)]
                        acc[idxr, pl.ds(di + kk * 16, 16)] = ad + e
                    return 0
                lax.fori_loop(0, G, row_body, 0)
                return 0
            lax.fori_loop(0, nch, chunk_body, 0)

            for z in range(RNG // 8):
                pltpu.sync_copy(acc.at[pl.ds(z * 8, 8)],
                                out_hbm.at[pl.ds(node0 + z * 8, 8)])

    return k(srcp, dstp, xp, t, starts)


def kernel(x, edge_index, params):
    src = edge_index[0]
    dst = edge_index[1]
    order = jnp.argsort(dst)
    dsts = dst[order]
    srcp = jnp.concatenate([src[order], jnp.zeros((EP - E,), jnp.int32)])
    dstp = jnp.concatenate([dsts, jnp.full((EP - E,), N, jnp.int32)])
    bounds = jnp.arange(0, (NRANGES + 1) * RNG, RNG)
    starts = jnp.searchsorted(dsts, bounds).astype(jnp.int32)
    starts = jnp.concatenate(
        [starts, jnp.full((STN - NRANGES - 1,), E, jnp.int32)])

    h = x
    dims = [(128, 512), (512, 256), (256, 128)]
    for l, (di, ho) in enumerate(dims):
        xp = _mm_bias_relu(h, params[f"Wp{l}"],
                           params[f"bp{l}"].reshape(1, di), di)
        numden = _edge_softmax_aggr(srcp, dstp, xp,
                                    params[f"t{l}"].reshape(di), starts, di)
        Wrl = params[f"Wr{l}"] + params[f"Wlin{l}"]
        bias = (params[f"bl{l}"] + params[f"blin{l}"]).reshape(1, ho)
        h = _combine(numden, h, params[f"Wl{l}"], Wrl, bias, di, ho,
                     relu=(l < 2))
    return h
